# Initial kernel scaffold; baseline (speedup 1.0000x reference)
import jax
import jax.numpy as jnp
from jax.experimental import pallas as pl


def kernel(state, action, tar_scores, params):
    raise NotImplementedError("write your pallas kernel here")


# speedup vs baseline: 2.1653x; 2.1653x over previous
import functools

import jax
import jax.numpy as jnp
from jax.experimental import pallas as pl

BS = 500
NOBJ = 50
K = 16
HID = 64
EMB = 32
NCLS = 3
N = BS * NOBJ

G = 4


def _mm(a, b):
    return jax.lax.dot_general(
        a, b, (((1,), (0,)), ((), ())),
        precision=jax.lax.Precision.DEFAULT,
        preferred_element_type=jnp.float32)


def _branch_front(si, oh, w):
    t = jnp.tanh(_mm(si, w['spW1']) + w['spb1'])
    sp = _mm(t, w['spW2']) + w['spb2']
    ce = (oh[:, 0:1] * w['E'][0:1, :] + oh[:, 1:2] * w['E'][1:2, :]
          + oh[:, 2:3] * w['E'][2:3, :])
    return jnp.tanh(jnp.concatenate([sp, ce], axis=1))


def _branch_edges_tail(h, pens, w):
    aggs = []
    for g in range(G):
        xg = h[g * NOBJ:(g + 1) * NOBJ]
        xi = jax.lax.broadcast_in_dim(
            xg, (NOBJ, NOBJ, HID + EMB), (0, 2)).reshape(NOBJ * NOBJ,
                                                         HID + EMB)
        xj = jax.lax.broadcast_in_dim(
            xg, (NOBJ, NOBJ, HID + EMB), (1, 2)).reshape(NOBJ * NOBJ,
                                                         HID + EMB)
        e = jnp.concatenate([xi, xj - xi], axis=1)
        pre = jnp.tanh(_mm(e, w['mW1']) + w['mb1'])
        t = _mm(pre, w['mW2']) + (w['mb2'] + pens[g])
        aggs.append(t.reshape(NOBJ, NOBJ, HID).max(axis=1))
    x = jnp.tanh(jnp.concatenate(aggs, axis=0))
    t = jnp.tanh(_mm(x, w['tW1']) + w['tb1'])
    return _mm(t, w['tW2']) + w['tb2']


def _kernel(pos_ref, post_ref, act_ref, ts_ref, oh_ref,
            w1_refs, w2_refs, q1_ref, q2_ref):
    pos = pos_ref[...]
    post = post_ref[0]
    act = act_ref[...]
    ts = ts_ref[...]
    oh = oh_ref[...]

    ii = jax.lax.broadcasted_iota(jnp.int32, (NOBJ, NOBJ), 0)
    jj = jax.lax.broadcasted_iota(jnp.int32, (NOBJ, NOBJ), 1)
    eye_pen = jnp.where(ii == jj, jnp.float32(1e10), jnp.float32(0.0))
    d_list = []
    for g in range(G):
        sl = slice(g * NOBJ, (g + 1) * NOBJ)
        dx = pos[sl, 0:1] - post[0:1, sl]
        dy = pos[sl, 1:2] - post[1:2, sl]
        d_list.append(dx * dx + dy * dy + eye_pen)
    d_all = jnp.concatenate(d_list, axis=0)

    colidx = jax.lax.broadcasted_iota(jnp.int32, (G * NOBJ, NOBJ), 1)
    cur = d_all
    th = None
    for _ in range(K):
        th = jnp.min(cur, axis=1, keepdims=True)
        ismin = cur == th
        first = jnp.min(jnp.where(ismin, colidx, NOBJ + 1), axis=1,
                        keepdims=True)
        cur = jnp.where(colidx == first, jnp.float32(3e30), cur)
    def _flat_i(v):
        return jax.lax.broadcast_in_dim(
            v, (NOBJ, NOBJ, 1), (0, 2)).reshape(NOBJ * NOBJ, 1)

    def _flat_j(v):
        return jax.lax.broadcast_in_dim(
            v, (NOBJ, NOBJ, 1), (1, 2)).reshape(NOBJ * NOBJ, 1)

    i3 = jax.lax.broadcasted_iota(jnp.int32, (NOBJ, NOBJ, 1), 0)
    j3 = jax.lax.broadcasted_iota(jnp.int32, (NOBJ, NOBJ, 1), 1)
    eyef = jnp.where(i3 == j3, jnp.float32(1e10),
                     jnp.float32(0.0)).reshape(NOBJ * NOBJ, 1)
    pens = []
    for g in range(G):
        sl = slice(g * NOBJ, (g + 1) * NOBJ)
        dxf = _flat_i(pos[sl, 0:1]) - _flat_j(pos[sl, 0:1])
        dyf = _flat_i(pos[sl, 1:2]) - _flat_j(pos[sl, 1:2])
        d2f = dxf * dxf + dyf * dyf + eyef
        pens.append(jnp.where(d2f <= _flat_i(th[sl]), jnp.float32(0.0),
                              jnp.float32(-1e30)))

    si = jnp.concatenate([pos, act, ts], axis=1)
    for w, out_ref in ((w1_refs, q1_ref), (w2_refs, q2_ref)):
        h = _branch_front(si, oh, w)
        out_ref[...] = _branch_edges_tail(h, pens, w)


def _row_spec(d):
    return pl.BlockSpec((G * NOBJ, d), lambda i: (i, 0))


def _full_spec(shape):
    nd = len(shape)
    return pl.BlockSpec(shape, lambda i, nd=nd: (0,) * nd)


WKEYS = ('spW1', 'spb1', 'spW2', 'spb2', 'E', 'mW1', 'mb1',
         'mW2', 'mb2', 'tW1', 'tb1', 'tW2', 'tb2')


def _kernel_entry(pos, post, act, ts, oh, w1, w2):
    def body(pos_ref, post_ref, act_ref, ts_ref, oh_ref, *refs):
        n = len(WKEYS)
        w1_refs = dict(zip(WKEYS, refs[:n]))
        w2_refs = dict(zip(WKEYS, refs[n:2 * n]))
        w1v = {k: r[...] for k, r in w1_refs.items()}
        w2v = {k: r[...] for k, r in w2_refs.items()}
        _kernel(pos_ref, post_ref, act_ref, ts_ref, oh_ref,
                w1v, w2v, refs[2 * n], refs[2 * n + 1])

    in_specs = [
        _row_spec(2),
        pl.BlockSpec((1, 2, G * NOBJ), lambda i: (i, 0, 0)),
        _row_spec(2),
        _row_spec(2),
        _row_spec(3),
    ]
    flat_w = []
    for w in (w1, w2):
        for k in WKEYS:
            in_specs.append(_full_spec(w[k].shape))
            flat_w.append(w[k])
    return pl.pallas_call(
        body,
        grid=(BS // G,),
        in_specs=in_specs,
        out_specs=[_row_spec(1), _row_spec(1)],
        out_shape=[jax.ShapeDtypeStruct((N, 1), jnp.float32)] * 2,
    )(pos, post, act, ts, oh, *flat_w)


def _prep_weights(p, s):
    return {
        'spW1': p['sp%d_W1' % s],
        'spb1': p['sp%d_b1' % s].reshape(1, HID),
        'spW2': p['sp%d_W2' % s],
        'spb2': p['sp%d_b2' % s].reshape(1, HID),
        'E': jnp.tanh(p['emb%d_T' % s]) @ p['emb%d_W' % s]
             + p['emb%d_b' % s],
        'mW1': p['mlp%d_W1' % s],
        'mb1': p['mlp%d_b1' % s].reshape(1, HID),
        'mW2': p['mlp%d_W2' % s],
        'mb2': p['mlp%d_b2' % s].reshape(1, HID),
        'tW1': p['tail%d_W1' % s],
        'tb1': p['tail%d_b1' % s].reshape(1, HID),
        'tW2': p['tail%d_W2' % s],
        'tb2': p['tail%d_b2' % s].reshape(1, 1),
    }


@jax.jit
def kernel(state, action, tar_scores, params):
    st = state.reshape(BS, NOBJ, 3)
    pos = st[:, :, :2].reshape(N, 2)
    cat = st[:, :, 2].reshape(N).astype(jnp.int32)
    oh = jax.nn.one_hot(cat, NCLS, dtype=jnp.float32)
    act = action.reshape(N, 2)
    w1 = _prep_weights(params, 1)
    w2 = _prep_weights(params, 2)
    post3 = pos.T.reshape(2, BS // G, G * NOBJ).transpose(1, 0, 2)
    ts = jnp.tanh(tar_scores)
    q1, q2 = _kernel_entry(pos, post3, act, ts, oh, w1, w2)
    return q1.reshape(BS, NOBJ), q2.reshape(BS, NOBJ)


# speedup vs baseline: 2.7662x; 1.2775x over previous
import functools

import jax
import jax.numpy as jnp
from jax.experimental import pallas as pl

BS = 500
NOBJ = 50
K = 16
HID = 64
EMB = 32
NCLS = 3
N = BS * NOBJ

G = 8
NB = -(-BS // G) * G
NP = NB * NOBJ


def _mm(a, b):
    return jax.lax.dot_general(
        a, b, (((1,), (0,)), ((), ())),
        precision=jax.lax.Precision.DEFAULT,
        preferred_element_type=jnp.float32)


def _branch_front(si, oh, w):
    t = jnp.tanh(_mm(si, w['spW1']) + w['spb1'])
    sp = _mm(t, w['spW2']) + w['spb2']
    ce = (oh[:, 0:1] * w['E'][0:1, :] + oh[:, 1:2] * w['E'][1:2, :]
          + oh[:, 2:3] * w['E'][2:3, :])
    return jnp.tanh(jnp.concatenate([sp, ce], axis=1))


def _branch_edges_tail(h, pens, w):
    z32 = jnp.zeros((NOBJ * NOBJ, 32), jnp.float32)
    aggs = []
    for g in range(G):
        xg = h[g * NOBJ:(g + 1) * NOBJ]
        xi = jax.lax.broadcast_in_dim(
            xg, (NOBJ, NOBJ, HID + EMB), (0, 2)).reshape(NOBJ * NOBJ,
                                                         HID + EMB)
        xj = jax.lax.broadcast_in_dim(
            xg, (NOBJ, NOBJ, HID + EMB), (1, 2)).reshape(NOBJ * NOBJ,
                                                         HID + EMB)
        e = jnp.concatenate([xi, z32, xj - xi, z32], axis=1)
        pre = jnp.tanh(_mm(e, w['mW1p']) + w['mb1'])
        t = _mm(pre, w['mW2']) + w['mb2'] + pens[g]
        aggs.append(t.reshape(NOBJ, NOBJ, HID).max(axis=1))
    x = jnp.tanh(jnp.concatenate(aggs, axis=0))
    t = jnp.tanh(_mm(x, w['tW1']) + w['tb1'])
    return _mm(t, w['tW2']) + w['tb2']


def _kernel(pos_ref, post_ref, act_ref, ts_ref, oh_ref, m1_ref, m2_ref,
            w1_refs, w2_refs, q1_ref, q2_ref):
    pos = pos_ref[...]
    post = post_ref[0]
    act = act_ref[...]
    ts = ts_ref[...]
    oh = oh_ref[...]

    ii = jax.lax.broadcasted_iota(jnp.int32, (NOBJ, NOBJ), 0)
    jj = jax.lax.broadcasted_iota(jnp.int32, (NOBJ, NOBJ), 1)
    eye_pen = jnp.where(ii == jj, jnp.float32(1e10), jnp.float32(0.0))
    d_list = []
    for g in range(G):
        sl = slice(g * NOBJ, (g + 1) * NOBJ)
        dx = pos[sl, 0:1] - post[0:1, sl]
        dy = pos[sl, 1:2] - post[1:2, sl]
        d_list.append(dx * dx + dy * dy + eye_pen)
    d_all = jnp.concatenate(d_list, axis=0)

    colidx = jax.lax.broadcasted_iota(jnp.int32, (G * NOBJ, NOBJ), 1)
    cur = d_all
    th = None
    for _ in range(K):
        th = jnp.min(cur, axis=1, keepdims=True)
        ismin = cur == th
        first = jnp.min(jnp.where(ismin, colidx, NOBJ + 1), axis=1,
                        keepdims=True)
        cur = jnp.where(colidx == first, jnp.float32(3e30), cur)
    m1 = m1_ref[...]
    m2 = m2_ref[...]
    ones1 = jnp.ones((NOBJ, 1), jnp.float32)
    pens = []
    for g in range(G):
        sl = slice(g * NOBJ, (g + 1) * NOBJ)
        pen2d = jnp.where(d_all[sl] <= th[sl], jnp.float32(0.0),
                          jnp.float32(-1e30))
        pens.append(_mm(_mm(m1, pen2d) * m2, ones1))

    si = jnp.concatenate([pos, act, ts], axis=1)
    for w, out_ref in ((w1_refs, q1_ref), (w2_refs, q2_ref)):
        h = _branch_front(si, oh, w)
        out_ref[...] = _branch_edges_tail(h, pens, w)


def _row_spec(d):
    return pl.BlockSpec((G * NOBJ, d), lambda i: (i, 0))


def _full_spec(shape):
    nd = len(shape)
    return pl.BlockSpec(shape, lambda i, nd=nd: (0,) * nd)


WKEYS = ('spW1', 'spb1', 'spW2', 'spb2', 'E', 'mW1p', 'mb1',
         'mW2', 'mb2', 'tW1', 'tb1', 'tW2', 'tb2')


def _kernel_entry(pos, post, act, ts, oh, m1, m2, w1, w2):
    def body(pos_ref, post_ref, act_ref, ts_ref, oh_ref, m1_ref, m2_ref,
             *refs):
        n = len(WKEYS)
        w1_refs = dict(zip(WKEYS, refs[:n]))
        w2_refs = dict(zip(WKEYS, refs[n:2 * n]))
        w1v = {k: r[...] for k, r in w1_refs.items()}
        w2v = {k: r[...] for k, r in w2_refs.items()}
        _kernel(pos_ref, post_ref, act_ref, ts_ref, oh_ref, m1_ref,
                m2_ref, w1v, w2v, refs[2 * n], refs[2 * n + 1])

    in_specs = [
        _row_spec(2),
        pl.BlockSpec((1, 2, G * NOBJ), lambda i: (i, 0, 0)),
        _row_spec(2),
        _row_spec(2),
        _row_spec(3),
        _full_spec(m1.shape),
        _full_spec(m2.shape),
    ]
    flat_w = []
    for w in (w1, w2):
        for k in WKEYS:
            in_specs.append(_full_spec(w[k].shape))
            flat_w.append(w[k])
    return pl.pallas_call(
        body,
        grid=(NB // G,),
        in_specs=in_specs,
        out_specs=[_row_spec(1), _row_spec(1)],
        out_shape=[jax.ShapeDtypeStruct((NP, 1), jnp.float32)] * 2,
    )(pos, post, act, ts, oh, m1, m2, *flat_w)


def _prep_weights(p, s):
    return {
        'spW1': p['sp%d_W1' % s],
        'spb1': p['sp%d_b1' % s].reshape(1, HID),
        'spW2': p['sp%d_W2' % s],
        'spb2': p['sp%d_b2' % s].reshape(1, HID),
        'E': jnp.tanh(p['emb%d_T' % s]) @ p['emb%d_W' % s]
             + p['emb%d_b' % s],
        'mW1p': jnp.concatenate([
            p['mlp%d_W1' % s][:HID + EMB],
            jnp.zeros((32, HID), jnp.float32),
            p['mlp%d_W1' % s][HID + EMB:],
            jnp.zeros((32, HID), jnp.float32)], axis=0),
        'mb1': p['mlp%d_b1' % s].reshape(1, HID),
        'mW2': p['mlp%d_W2' % s],
        'mb2': p['mlp%d_b2' % s].reshape(1, HID),
        'tW1': p['tail%d_W1' % s],
        'tb1': p['tail%d_b1' % s].reshape(1, HID),
        'tW2': p['tail%d_W2' % s],
        'tb2': p['tail%d_b2' % s].reshape(1, 1),
    }


@jax.jit
def kernel(state, action, tar_scores, params):
    st = state.reshape(BS, NOBJ, 3)
    npad = NP - N

    def _pad(x):
        return jnp.pad(x, ((0, npad), (0, 0))) if npad else x

    pos = _pad(st[:, :, :2].reshape(N, 2))
    cat = st[:, :, 2].reshape(N).astype(jnp.int32)
    oh = _pad(jax.nn.one_hot(cat, NCLS, dtype=jnp.float32))
    act = _pad(action.reshape(N, 2))
    w1 = _prep_weights(params, 1)
    w2 = _prep_weights(params, 2)
    post3 = pos.T.reshape(2, NB // G, G * NOBJ).transpose(1, 0, 2)
    ts = _pad(jnp.tanh(tar_scores))
    r = jnp.arange(NOBJ * NOBJ)
    c = jnp.arange(NOBJ)
    m1 = (r[:, None] // NOBJ == c[None, :]).astype(jnp.float32)
    m2 = (r[:, None] % NOBJ == c[None, :]).astype(jnp.float32)
    q1, q2 = _kernel_entry(pos, post3, act, ts, oh, m1, m2, w1, w2)
    return q1[:N].reshape(BS, NOBJ), q2[:N].reshape(BS, NOBJ)


# speedup vs baseline: 3.5279x; 1.2754x over previous
import functools

import jax
import jax.numpy as jnp
from jax.experimental import pallas as pl

BS = 500
NOBJ = 50
K = 16
HID = 64
EMB = 32
NCLS = 3
N = BS * NOBJ

G = 8
NB = -(-BS // G) * G
NP = NB * NOBJ


def _mm(a, b):
    return jax.lax.dot_general(
        a, b, (((1,), (0,)), ((), ())),
        precision=jax.lax.Precision.DEFAULT,
        preferred_element_type=jnp.float32)


def _branch_front(si, oh, w):
    t = jnp.tanh(_mm(si, w['spW1']) + w['spb1'])
    sp = _mm(t, w['spW2']) + w['spb2']
    ce = (oh[:, 0:1] * w['E'][0:1, :] + oh[:, 1:2] * w['E'][1:2, :]
          + oh[:, 2:3] * w['E'][2:3, :])
    return jnp.tanh(jnp.concatenate([sp, ce], axis=1))


def _branch_edges_tail(h, pens, w):
    z32 = jnp.zeros((NOBJ * NOBJ, 32), jnp.float32)
    aggs = []
    for g in range(G):
        xg = h[g * NOBJ:(g + 1) * NOBJ]
        xi = jax.lax.broadcast_in_dim(
            xg, (NOBJ, NOBJ, HID + EMB), (0, 2)).reshape(NOBJ * NOBJ,
                                                         HID + EMB)
        xj = jax.lax.broadcast_in_dim(
            xg, (NOBJ, NOBJ, HID + EMB), (1, 2)).reshape(NOBJ * NOBJ,
                                                         HID + EMB)
        e = jnp.concatenate([xi, z32, xj - xi, z32], axis=1)
        pre = jnp.tanh(_mm(e, w['mW1p']) + w['mb1'])
        t = _mm(pre, w['mW2']) + w['mb2'] + pens[g]
        aggs.append(t.reshape(NOBJ, NOBJ, HID).max(axis=1))
    x = jnp.tanh(jnp.concatenate(aggs, axis=0))
    t = jnp.tanh(_mm(x, w['tW1']) + w['tb1'])
    return _mm(t, w['tW2']) + w['tb2']


def _kernel(pos_ref, post_ref, act_ref, ts_ref, oh_ref, m1_ref, m2_ref,
            w1_refs, w2_refs, q1_ref, q2_ref):
    pos = pos_ref[...]
    post = post_ref[0]
    act = act_ref[...]
    ts = ts_ref[...]
    oh = oh_ref[...]

    ii = jax.lax.broadcasted_iota(jnp.int32, (NOBJ, NOBJ), 0)
    jj = jax.lax.broadcasted_iota(jnp.int32, (NOBJ, NOBJ), 1)
    eye_pen = jnp.where(ii == jj, jnp.float32(1e10), jnp.float32(0.0))
    d_list = []
    for g in range(G):
        sl = slice(g * NOBJ, (g + 1) * NOBJ)
        dx = pos[sl, 0:1] - post[0:1, sl]
        dy = pos[sl, 1:2] - post[1:2, sl]
        d_list.append(dx * dx + dy * dy + eye_pen)

    d_wide = jnp.concatenate(d_list, axis=1)
    rowidx = jax.lax.broadcasted_iota(jnp.int32, (NOBJ, G * NOBJ), 0)
    cur = d_wide
    th = None
    for _ in range(K):
        th = jnp.min(cur, axis=0, keepdims=True)
        ismin = cur == th
        first = jnp.min(jnp.where(ismin, rowidx, NOBJ + 1), axis=0,
                        keepdims=True)
        cur = jnp.where(rowidx == first, jnp.float32(3e30), cur)
    m1 = m1_ref[...]
    m2 = m2_ref[...]
    ones1 = jnp.ones((NOBJ, 1), jnp.float32)
    pens = []
    for g in range(G):
        sl = slice(g * NOBJ, (g + 1) * NOBJ)
        pen2dt = jnp.where(d_list[g] <= th[0:1, sl], jnp.float32(0.0),
                           jnp.float32(-1e30))
        pens.append(_mm(_mm(m2, pen2dt) * m1, ones1))

    si = jnp.concatenate([pos, act, ts], axis=1)
    for w, out_ref in ((w1_refs, q1_ref), (w2_refs, q2_ref)):
        h = _branch_front(si, oh, w)
        out_ref[...] = _branch_edges_tail(h, pens, w)


def _row_spec(d):
    return pl.BlockSpec((G * NOBJ, d), lambda i: (i, 0))


def _full_spec(shape):
    nd = len(shape)
    return pl.BlockSpec(shape, lambda i, nd=nd: (0,) * nd)


WKEYS = ('spW1', 'spb1', 'spW2', 'spb2', 'E', 'mW1p', 'mb1',
         'mW2', 'mb2', 'tW1', 'tb1', 'tW2', 'tb2')


def _kernel_entry(pos, post, act, ts, oh, m1, m2, w1, w2):
    def body(pos_ref, post_ref, act_ref, ts_ref, oh_ref, m1_ref, m2_ref,
             *refs):
        n = len(WKEYS)
        w1_refs = dict(zip(WKEYS, refs[:n]))
        w2_refs = dict(zip(WKEYS, refs[n:2 * n]))
        w1v = {k: r[...] for k, r in w1_refs.items()}
        w2v = {k: r[...] for k, r in w2_refs.items()}
        _kernel(pos_ref, post_ref, act_ref, ts_ref, oh_ref, m1_ref,
                m2_ref, w1v, w2v, refs[2 * n], refs[2 * n + 1])

    in_specs = [
        _row_spec(2),
        pl.BlockSpec((1, 2, G * NOBJ), lambda i: (i, 0, 0)),
        _row_spec(2),
        _row_spec(2),
        _row_spec(3),
        _full_spec(m1.shape),
        _full_spec(m2.shape),
    ]
    flat_w = []
    for w in (w1, w2):
        for k in WKEYS:
            in_specs.append(_full_spec(w[k].shape))
            flat_w.append(w[k])
    return pl.pallas_call(
        body,
        grid=(NB // G,),
        in_specs=in_specs,
        out_specs=[_row_spec(1), _row_spec(1)],
        out_shape=[jax.ShapeDtypeStruct((NP, 1), jnp.float32)] * 2,
    )(pos, post, act, ts, oh, m1, m2, *flat_w)


def _prep_weights(p, s):
    return {
        'spW1': p['sp%d_W1' % s],
        'spb1': p['sp%d_b1' % s].reshape(1, HID),
        'spW2': p['sp%d_W2' % s],
        'spb2': p['sp%d_b2' % s].reshape(1, HID),
        'E': jnp.tanh(p['emb%d_T' % s]) @ p['emb%d_W' % s]
             + p['emb%d_b' % s],
        'mW1p': jnp.concatenate([
            p['mlp%d_W1' % s][:HID + EMB],
            jnp.zeros((32, HID), jnp.float32),
            p['mlp%d_W1' % s][HID + EMB:],
            jnp.zeros((32, HID), jnp.float32)], axis=0),
        'mb1': p['mlp%d_b1' % s].reshape(1, HID),
        'mW2': p['mlp%d_W2' % s],
        'mb2': p['mlp%d_b2' % s].reshape(1, HID),
        'tW1': p['tail%d_W1' % s],
        'tb1': p['tail%d_b1' % s].reshape(1, HID),
        'tW2': p['tail%d_W2' % s],
        'tb2': p['tail%d_b2' % s].reshape(1, 1),
    }


@jax.jit
def kernel(state, action, tar_scores, params):
    st = state.reshape(BS, NOBJ, 3)
    npad = NP - N

    def _pad(x):
        return jnp.pad(x, ((0, npad), (0, 0))) if npad else x

    pos = _pad(st[:, :, :2].reshape(N, 2))
    cat = st[:, :, 2].reshape(N).astype(jnp.int32)
    oh = _pad(jax.nn.one_hot(cat, NCLS, dtype=jnp.float32))
    act = _pad(action.reshape(N, 2))
    w1 = _prep_weights(params, 1)
    w2 = _prep_weights(params, 2)
    post3 = pos.T.reshape(2, NB // G, G * NOBJ).transpose(1, 0, 2)
    ts = _pad(jnp.tanh(tar_scores))
    r = jnp.arange(NOBJ * NOBJ)
    c = jnp.arange(NOBJ)
    m1 = (r[:, None] // NOBJ == c[None, :]).astype(jnp.float32)
    m2 = (r[:, None] % NOBJ == c[None, :]).astype(jnp.float32)
    q1, q2 = _kernel_entry(pos, post3, act, ts, oh, m1, m2, w1, w2)
    return q1[:N].reshape(BS, NOBJ), q2[:N].reshape(BS, NOBJ)


# speedup vs baseline: 4.8557x; 1.3764x over previous
import jax
import jax.numpy as jnp
from jax.experimental import pallas as pl

BS = 500
NOBJ = 50
K = 16
HID = 64
EMB = 32
NCLS = 3
N = BS * NOBJ

G = 8
NR = 64
IB = 7
NE = IB * NOBJ * 8
NB = -(-BS // G) * G
NRW = NB * NR


def _mm(a, b):
    return jax.lax.dot_general(
        a, b, (((1,), (0,)), ((), ())),
        precision=jax.lax.Precision.DEFAULT,
        preferred_element_type=jnp.float32)


def _branch_front(si, oh, w):
    t = jnp.tanh(_mm(si, w['spW1']) + w['spb1'])
    sp = _mm(t, w['spW2']) + w['spb2']
    ce = (oh[:, 0:1] * w['E'][0:1, :] + oh[:, 1:2] * w['E'][1:2, :]
          + oh[:, 2:3] * w['E'][2:3, :])
    return jnp.tanh(jnp.concatenate([sp, ce], axis=1))


def _branch_edges_tail(h, pens, w):
    z32 = jnp.zeros((NE, 32), jnp.float32)
    zpad = jnp.zeros((NR - IB * 8, HID), jnp.float32)
    aggs = []
    for g in range(G):
        xg = h[g * NR:(g + 1) * NR]
        xjp = jax.lax.broadcast_in_dim(
            xg[:NOBJ], (NOBJ, 8, HID + EMB), (0, 2)).reshape(
                NOBJ * 8, HID + EMB)
        xi_all = jnp.concatenate([
            jax.lax.broadcast_in_dim(
                xg[8 * ib:8 * ib + 8], (NOBJ, 8, HID + EMB),
                (1, 2)).reshape(NOBJ * 8, HID + EMB)
            for ib in range(IB)], axis=0)
        xj_all = jnp.concatenate([xjp] * IB, axis=0)
        e = jnp.concatenate([xi_all, z32, xj_all - xi_all, z32], axis=1)
        pre = jnp.tanh(_mm(e, w['mW1p']) + w['mb1'])
        t = _mm(pre, w['mW2']) + w['mb2'] + pens[g]
        agg = jnp.concatenate(
            [t[ib * NOBJ * 8:(ib + 1) * NOBJ * 8].reshape(
                NOBJ, 8, HID).max(axis=0) for ib in range(IB)]
            + [zpad], axis=0)
        aggs.append(agg)
    x = jnp.tanh(jnp.concatenate(aggs, axis=0))
    t = jnp.tanh(_mm(x, w['tW1']) + w['tb1'])
    return _mm(t, w['tW2']) + w['tb2']


def _kernel(pos_ref, post_ref, act_ref, ts_ref, oh_ref, mi_ref, mj_ref,
            w1_refs, w2_refs, q1_ref, q2_ref):
    pos = pos_ref[...]
    post = post_ref[0]
    act = act_ref[...]
    ts = ts_ref[...]
    oh = oh_ref[...]

    jj = jax.lax.broadcasted_iota(jnp.int32, (NOBJ, NR), 0)
    ii = jax.lax.broadcasted_iota(jnp.int32, (NOBJ, NR), 1)
    eye_pen = jnp.where(ii == jj, jnp.float32(1e10), jnp.float32(0.0))
    d_list = []
    for g in range(G):
        cs = slice(g * NR, g * NR + NOBJ)
        ns = slice(g * NR, (g + 1) * NR)
        dx = pos[cs, 0:1] - post[0:1, ns]
        dy = pos[cs, 1:2] - post[1:2, ns]
        d_list.append(dx * dx + dy * dy + eye_pen)

    d_wide = jnp.concatenate(d_list, axis=1)
    rowidx = jax.lax.broadcasted_iota(jnp.int32, (NOBJ, G * NR), 0)
    cur = d_wide
    th = None
    for _ in range(K):
        th = jnp.min(cur, axis=0, keepdims=True)
        ismin = cur == th
        first = jnp.min(jnp.where(ismin, rowidx, NOBJ + 1), axis=0,
                        keepdims=True)
        cur = jnp.where(rowidx == first, jnp.float32(3e30), cur)
    mi = mi_ref[...]
    mj = mj_ref[...]
    ones1 = jnp.ones((NR, 1), jnp.float32)
    pens = []
    for g in range(G):
        sl = slice(g * NR, (g + 1) * NR)
        pen2dt = jnp.where(d_list[g] <= th[0:1, sl], jnp.float32(0.0),
                           jnp.float32(-1e30))
        pens.append(_mm(_mm(mj, pen2dt) * mi, ones1))

    si = jnp.concatenate([pos, act, ts], axis=1)
    for w, out_ref in ((w1_refs, q1_ref), (w2_refs, q2_ref)):
        h = _branch_front(si, oh, w)
        out_ref[...] = _branch_edges_tail(h, pens, w)


def _row_spec(d):
    return pl.BlockSpec((G * NR, d), lambda i: (i, 0))


def _full_spec(shape):
    nd = len(shape)
    return pl.BlockSpec(shape, lambda i, nd=nd: (0,) * nd)


WKEYS = ('spW1', 'spb1', 'spW2', 'spb2', 'E', 'mW1p', 'mb1',
         'mW2', 'mb2', 'tW1', 'tb1', 'tW2', 'tb2')


def _kernel_entry(pos, post, act, ts, oh, mi, mj, w1, w2):
    def body(pos_ref, post_ref, act_ref, ts_ref, oh_ref, mi_ref, mj_ref,
             *refs):
        n = len(WKEYS)
        w1_refs = dict(zip(WKEYS, refs[:n]))
        w2_refs = dict(zip(WKEYS, refs[n:2 * n]))
        w1v = {k: r[...] for k, r in w1_refs.items()}
        w2v = {k: r[...] for k, r in w2_refs.items()}
        _kernel(pos_ref, post_ref, act_ref, ts_ref, oh_ref, mi_ref,
                mj_ref, w1v, w2v, refs[2 * n], refs[2 * n + 1])

    in_specs = [
        _row_spec(2),
        pl.BlockSpec((1, 2, G * NR), lambda i: (i, 0, 0)),
        _row_spec(2),
        _row_spec(2),
        _row_spec(3),
        _full_spec(mi.shape),
        _full_spec(mj.shape),
    ]
    flat_w = []
    for w in (w1, w2):
        for k in WKEYS:
            in_specs.append(_full_spec(w[k].shape))
            flat_w.append(w[k])
    return pl.pallas_call(
        body,
        grid=(NB // G,),
        in_specs=in_specs,
        out_specs=[_row_spec(1), _row_spec(1)],
        out_shape=[jax.ShapeDtypeStruct((NRW, 1), jnp.float32)] * 2,
    )(pos, post, act, ts, oh, mi, mj, *flat_w)


def _prep_weights(p, s):
    return {
        'spW1': p['sp%d_W1' % s],
        'spb1': p['sp%d_b1' % s].reshape(1, HID),
        'spW2': p['sp%d_W2' % s],
        'spb2': p['sp%d_b2' % s].reshape(1, HID),
        'E': jnp.tanh(p['emb%d_T' % s]) @ p['emb%d_W' % s]
             + p['emb%d_b' % s],
        'mW1p': jnp.concatenate([
            p['mlp%d_W1' % s][:HID + EMB],
            jnp.zeros((32, HID), jnp.float32),
            p['mlp%d_W1' % s][HID + EMB:],
            jnp.zeros((32, HID), jnp.float32)], axis=0),
        'mb1': p['mlp%d_b1' % s].reshape(1, HID),
        'mW2': p['mlp%d_W2' % s],
        'mb2': p['mlp%d_b2' % s].reshape(1, HID),
        'tW1': p['tail%d_W1' % s],
        'tb1': p['tail%d_b1' % s].reshape(1, HID),
        'tW2': p['tail%d_W2' % s],
        'tb2': p['tail%d_b2' % s].reshape(1, 1),
    }


@jax.jit
def kernel(state, action, tar_scores, params):
    st = state.reshape(BS, NOBJ, 3)

    def _pad(x):
        d = x.shape[1]
        xb = x.reshape(BS, NOBJ, d)
        xb = jnp.pad(xb, ((0, NB - BS), (0, NR - NOBJ), (0, 0)))
        return xb.reshape(NRW, d)

    pos = _pad(st[:, :, :2].reshape(N, 2))
    cat = st[:, :, 2].reshape(N).astype(jnp.int32)
    oh = _pad(jax.nn.one_hot(cat, NCLS, dtype=jnp.float32))
    act = _pad(action.reshape(N, 2))
    ts = _pad(jnp.tanh(tar_scores))
    w1 = _prep_weights(params, 1)
    w2 = _prep_weights(params, 2)
    post3 = pos.T.reshape(2, NB // G, G * NR).transpose(1, 0, 2)
    r = jnp.arange(NE)
    i_of_r = 8 * (r // (NOBJ * 8)) + r % 8
    j_of_r = (r % (NOBJ * 8)) // 8
    mi = (i_of_r[:, None] == jnp.arange(NR)[None, :]).astype(jnp.float32)
    mj = (j_of_r[:, None] == jnp.arange(NOBJ)[None, :]).astype(
        jnp.float32)
    q1, q2 = _kernel_entry(pos, post3, act, ts, oh, mi, mj, w1, w2)
    q1 = q1.reshape(NB, NR)[:BS, :NOBJ]
    q2 = q2.reshape(NB, NR)[:BS, :NOBJ]
    return q1, q2


# speedup vs baseline: 5.2647x; 1.0842x over previous
import jax
import jax.numpy as jnp
from jax.experimental import pallas as pl

BS = 500
NOBJ = 50
K = 16
HID = 64
EMB = 32
NCLS = 3
N = BS * NOBJ

G = 8
NR = 64
IB = 7
NE = IB * NOBJ * 8
NB = -(-BS // G) * G
NRW = NB * NR


def _mm(a, b):
    return jax.lax.dot_general(
        a, b, (((1,), (0,)), ((), ())),
        precision=jax.lax.Precision.DEFAULT,
        preferred_element_type=jnp.float32)


def _branch_front(si, oh, w):
    t = jnp.tanh(_mm(si, w['spW1']) + w['spb1'])
    sp = _mm(t, w['spW2']) + w['spb2']
    ce = (oh[:, 0:1] * w['E'][0:1, :] + oh[:, 1:2] * w['E'][1:2, :]
          + oh[:, 2:3] * w['E'][2:3, :])
    return jnp.tanh(jnp.concatenate([sp, ce], axis=1))


def _branch_edges_tail(h, pens, w):
    z32 = jnp.zeros((NE, 32), jnp.float32)
    zpad = jnp.zeros((NR - IB * 8, HID), jnp.float32)
    aggs = []
    for g in range(G):
        xg = h[g * NR:(g + 1) * NR]
        xjp = jax.lax.broadcast_in_dim(
            xg[:NOBJ], (NOBJ, 8, HID + EMB), (0, 2)).reshape(
                NOBJ * 8, HID + EMB)
        xi_all = jnp.concatenate([
            jax.lax.broadcast_in_dim(
                xg[8 * ib:8 * ib + 8], (NOBJ, 8, HID + EMB),
                (1, 2)).reshape(NOBJ * 8, HID + EMB)
            for ib in range(IB)], axis=0)
        xj_all = jnp.concatenate([xjp] * IB, axis=0)
        e = jnp.concatenate([xi_all, z32, xj_all - xi_all, z32],
                            axis=1).astype(jnp.bfloat16)
        pre = jnp.tanh(_mm(e, w['mW1p']) + w['mb1'])
        t = _mm(pre.astype(jnp.bfloat16), w['mW2']) + w['mb2'] + pens[g]
        agg = jnp.concatenate(
            [t[ib * NOBJ * 8:(ib + 1) * NOBJ * 8].reshape(
                NOBJ, 8, HID).max(axis=0) for ib in range(IB)]
            + [zpad], axis=0)
        aggs.append(agg)
    x = jnp.tanh(jnp.concatenate(aggs, axis=0))
    t = jnp.tanh(_mm(x, w['tW1']) + w['tb1'])
    return _mm(t, w['tW2']) + w['tb2']


def _kernel(pos_ref, post_ref, act_ref, ts_ref, oh_ref, mi_ref, mj_ref,
            w1_refs, w2_refs, q1_ref, q2_ref):
    pos = pos_ref[...]
    post = post_ref[0]
    act = act_ref[...]
    ts = ts_ref[...]
    oh = oh_ref[...]

    jj = jax.lax.broadcasted_iota(jnp.int32, (NOBJ, NR), 0)
    ii = jax.lax.broadcasted_iota(jnp.int32, (NOBJ, NR), 1)
    eye_pen = jnp.where(ii == jj, jnp.float32(1e10), jnp.float32(0.0))
    d_list = []
    for g in range(G):
        cs = slice(g * NR, g * NR + NOBJ)
        ns = slice(g * NR, (g + 1) * NR)
        dx = pos[cs, 0:1] - post[0:1, ns]
        dy = pos[cs, 1:2] - post[1:2, ns]
        d_list.append(dx * dx + dy * dy + eye_pen)

    d_wide = jnp.concatenate(d_list, axis=1)
    rowidx = jax.lax.broadcasted_iota(jnp.int32, (NOBJ, G * NR), 0)
    cur = d_wide
    th = None
    for _ in range(K):
        th = jnp.min(cur, axis=0, keepdims=True)
        ismin = cur == th
        first = jnp.min(jnp.where(ismin, rowidx, NOBJ + 1), axis=0,
                        keepdims=True)
        cur = jnp.where(rowidx == first, jnp.float32(3e30), cur)
    mi = mi_ref[...]
    mj = mj_ref[...]
    ones1 = jnp.ones((NR, 1), jnp.float32)
    pens = []
    for g in range(G):
        sl = slice(g * NR, (g + 1) * NR)
        pen2dt = jnp.where(d_list[g] <= th[0:1, sl], jnp.float32(0.0),
                           jnp.float32(-1e30))
        pens.append(_mm(_mm(mj, pen2dt) * mi, ones1))

    si = jnp.concatenate([pos, act, ts], axis=1)
    for w, out_ref in ((w1_refs, q1_ref), (w2_refs, q2_ref)):
        h = _branch_front(si, oh, w)
        out_ref[...] = _branch_edges_tail(h, pens, w)


def _row_spec(d):
    return pl.BlockSpec((G * NR, d), lambda i: (i, 0))


def _full_spec(shape):
    nd = len(shape)
    return pl.BlockSpec(shape, lambda i, nd=nd: (0,) * nd)


WKEYS = ('spW1', 'spb1', 'spW2', 'spb2', 'E', 'mW1p', 'mb1',
         'mW2', 'mb2', 'tW1', 'tb1', 'tW2', 'tb2')


def _kernel_entry(pos, post, act, ts, oh, mi, mj, w1, w2):
    def body(pos_ref, post_ref, act_ref, ts_ref, oh_ref, mi_ref, mj_ref,
             *refs):
        n = len(WKEYS)
        w1_refs = dict(zip(WKEYS, refs[:n]))
        w2_refs = dict(zip(WKEYS, refs[n:2 * n]))
        w1v = {k: r[...] for k, r in w1_refs.items()}
        w2v = {k: r[...] for k, r in w2_refs.items()}
        _kernel(pos_ref, post_ref, act_ref, ts_ref, oh_ref, mi_ref,
                mj_ref, w1v, w2v, refs[2 * n], refs[2 * n + 1])

    in_specs = [
        _row_spec(2),
        pl.BlockSpec((1, 2, G * NR), lambda i: (i, 0, 0)),
        _row_spec(2),
        _row_spec(2),
        _row_spec(3),
        _full_spec(mi.shape),
        _full_spec(mj.shape),
    ]
    flat_w = []
    for w in (w1, w2):
        for k in WKEYS:
            in_specs.append(_full_spec(w[k].shape))
            flat_w.append(w[k])
    return pl.pallas_call(
        body,
        grid=(NB // G,),
        in_specs=in_specs,
        out_specs=[_row_spec(1), _row_spec(1)],
        out_shape=[jax.ShapeDtypeStruct((NRW, 1), jnp.float32)] * 2,
    )(pos, post, act, ts, oh, mi, mj, *flat_w)


def _prep_weights(p, s):
    return {
        'spW1': p['sp%d_W1' % s],
        'spb1': p['sp%d_b1' % s].reshape(1, HID),
        'spW2': p['sp%d_W2' % s],
        'spb2': p['sp%d_b2' % s].reshape(1, HID),
        'E': jnp.tanh(p['emb%d_T' % s]) @ p['emb%d_W' % s]
             + p['emb%d_b' % s],
        'mW1p': jnp.concatenate([
            p['mlp%d_W1' % s][:HID + EMB],
            jnp.zeros((32, HID), jnp.float32),
            p['mlp%d_W1' % s][HID + EMB:],
            jnp.zeros((32, HID), jnp.float32)],
            axis=0).astype(jnp.bfloat16),
        'mb1': p['mlp%d_b1' % s].reshape(1, HID),
        'mW2': p['mlp%d_W2' % s].astype(jnp.bfloat16),
        'mb2': p['mlp%d_b2' % s].reshape(1, HID),
        'tW1': p['tail%d_W1' % s],
        'tb1': p['tail%d_b1' % s].reshape(1, HID),
        'tW2': p['tail%d_W2' % s],
        'tb2': p['tail%d_b2' % s].reshape(1, 1),
    }


@jax.jit
def kernel(state, action, tar_scores, params):
    st = state.reshape(BS, NOBJ, 3)

    def _pad(x):
        d = x.shape[1]
        xb = x.reshape(BS, NOBJ, d)
        xb = jnp.pad(xb, ((0, NB - BS), (0, NR - NOBJ), (0, 0)))
        return xb.reshape(NRW, d)

    pos = _pad(st[:, :, :2].reshape(N, 2))
    cat = st[:, :, 2].reshape(N).astype(jnp.int32)
    oh = _pad(jax.nn.one_hot(cat, NCLS, dtype=jnp.float32))
    act = _pad(action.reshape(N, 2))
    ts = _pad(jnp.tanh(tar_scores))
    w1 = _prep_weights(params, 1)
    w2 = _prep_weights(params, 2)
    post3 = pos.T.reshape(2, NB // G, G * NR).transpose(1, 0, 2)
    r = jnp.arange(NE)
    i_of_r = 8 * (r // (NOBJ * 8)) + r % 8
    j_of_r = (r % (NOBJ * 8)) // 8
    mi = (i_of_r[:, None] == jnp.arange(NR)[None, :]).astype(jnp.float32)
    mj = (j_of_r[:, None] == jnp.arange(NOBJ)[None, :]).astype(
        jnp.float32)
    q1, q2 = _kernel_entry(pos, post3, act, ts, oh, mi, mj, w1, w2)
    q1 = q1.reshape(NB, NR)[:BS, :NOBJ]
    q2 = q2.reshape(NB, NR)[:BS, :NOBJ]
    return q1, q2
